# baseline (speedup 1.0000x reference)
import jax
import jax.numpy as jnp
from jax.experimental import pallas as pl
from jax.experimental.pallas import tpu as pltpu


def _combine_body(t_ref, ac_ref, om_ref, xlo_ref, xhi_ref, nlo_ref, nhi_ref, o_ref):
    b = pl.program_id(0)
    tt = t_ref[b]
    c1 = ac_ref[tt]
    c2 = om_ref[tt]
    o_ref[:, :, :256] = c1 * xlo_ref[...] + c2 * nlo_ref[...]
    o_ref[:, :, 256:] = c1 * xhi_ref[...] + c2 * nhi_ref[...]


def kernel(x_start, t, noise, sqrt_alphas_cumprod, sqrt_one_minus_alphas_cumprod):
    B, C, H, W = x_start.shape
    HH = H // 2

    smem = pl.BlockSpec(memory_space=pltpu.SMEM)
    lo = pl.BlockSpec((1, C, HH, W), lambda b: (b, 0, 0, 0))
    hi = pl.BlockSpec((1, C, HH, W), lambda b: (b, 0, 1, 0))
    blk = pl.BlockSpec((1, C, H, W), lambda b: (b, 0, 0, 0))

    out = pl.pallas_call(
        _combine_body,
        grid=(B,),
        in_specs=[smem, smem, smem, lo, hi, lo, hi],
        out_specs=blk,
        out_shape=jax.ShapeDtypeStruct((B, C, H, W), jnp.float32),
    )(t.astype(jnp.int32), sqrt_alphas_cumprod, sqrt_one_minus_alphas_cumprod,
      x_start, x_start, noise, noise)
    return out
